# Initial kernel scaffold; baseline (speedup 1.0000x reference)
import jax
import jax.numpy as jnp
from jax.experimental import pallas as pl


def kernel(x, table, W1, b1, W2, b2):
    raise NotImplementedError("write your pallas kernel here")


# speedup vs baseline: 7.5147x; 7.5147x over previous
import functools

import jax
import jax.numpy as jnp
from jax import lax
from jax.experimental import pallas as pl
from jax.experimental.pallas import tpu as pltpu
from jax.experimental.pallas import tpu_sc as plsc

VOCAB = 100000
EMBED = 100
EPAD = 112
BATCH = 4096
SEQ = 200
NLANES = 16
NVEC = EPAD // NLANES

NC, NS = 2, 16
NW = NC * NS
BPW = BATCH // NW
NCH = 5
CH = SEQ // NCH

_mesh = plsc.VectorSubcoreMesh(core_axis_name="c", subcore_axis_name="s")


@functools.partial(
    pl.kernel,
    out_type=jax.ShapeDtypeStruct((BATCH, EPAD), jnp.float32),
    mesh=_mesh,
    scratch_types=[
        pltpu.VMEM((BPW * SEQ,), jnp.int32),
        pltpu.VMEM((SEQ, EPAD), jnp.float32),
        pltpu.VMEM((SEQ, EPAD), jnp.float32),
        pltpu.VMEM((BPW, EPAD), jnp.float32),
        pltpu.SemaphoreType.DMA,
        pltpu.SemaphoreType.DMA,
        pltpu.SemaphoreType.DMA,
    ],
    compiler_params=pltpu.CompilerParams(use_tc_tiling_on_sc=False),
)
def _pool_sc(x_hbm, tab_hbm, out_hbm, idx_v, rows_a, rows_b, acc_v,
             sem_a, sem_b, sem_i):
    wid = lax.axis_index("s") * NC + lax.axis_index("c")
    base = wid * (BPW * SEQ)
    pltpu.async_copy(x_hbm.at[pl.ds(base, BPW * SEQ)], idx_v, sem_i).wait()

    def fire(r, buf, sem):
        rb = pl.multiple_of(r * SEQ, SEQ)
        for j in range(NCH):
            pltpu.make_async_copy(
                tab_hbm.at[idx_v.at[pl.ds(rb + j * CH, CH)]],
                buf.at[pl.ds(j * CH, CH)],
                sem,
            ).start()

    def drain(r, buf, sem):
        rb = pl.multiple_of(r * SEQ, SEQ)
        for j in range(NCH):
            pltpu.make_async_copy(
                tab_hbm.at[idx_v.at[pl.ds(rb + j * CH, CH)]],
                buf.at[pl.ds(j * CH, CH)],
                sem,
            ).wait()

    def accum(r, buf):
        def body(s, carry):
            return tuple(
                carry[j] + buf[s, pl.ds(j * NLANES, NLANES)]
                for j in range(NVEC)
            )
        acc = lax.fori_loop(
            0, SEQ, body,
            tuple(jnp.zeros((NLANES,), jnp.float32) for _ in range(NVEC)),
        )
        for j in range(NVEC):
            acc_v[r, pl.ds(j * NLANES, NLANES)] = acc[j]

    fire(0, rows_a, sem_a)

    @pl.loop(0, BPW // 2)
    def _(i):
        r0 = i * 2
        r1 = r0 + 1
        fire(r1, rows_b, sem_b)
        drain(r0, rows_a, sem_a)
        accum(r0, rows_a)

        @pl.when(i < BPW // 2 - 1)
        def _():
            fire(r0 + 2, rows_a, sem_a)

        drain(r1, rows_b, sem_b)
        accum(r1, rows_b)

    pltpu.sync_copy(acc_v, out_hbm.at[pl.ds(wid * BPW, BPW)])


def _mlp_tc(x_ref, sums_ref, t0_ref, w1_ref, b1_ref, w2_ref, b2_ref, out_ref):
    n1 = jnp.sum((x_ref[...] != 0).astype(jnp.float32), axis=1, keepdims=True)
    s = sums_ref[...] - (float(SEQ) - n1) * t0_ref[...]
    h = s / (n1 + 1e-9)
    z = jnp.dot(h, w1_ref[...], preferred_element_type=jnp.float32)
    z = jnp.maximum(z + b1_ref[...], 0.0)
    out_ref[...] = (
        jnp.dot(z, w2_ref[...], preferred_element_type=jnp.float32)
        + b2_ref[...]
    )


@jax.jit
def kernel(x, table, W1, b1, W2, b2):
    tp = jnp.pad(table, ((0, 0), (0, EPAD - EMBED)))
    sums = _pool_sc(x.reshape(-1), tp)
    w1t = jnp.pad(W1, ((0, 0), (0, EPAD - EMBED))).T
    out = pl.pallas_call(
        _mlp_tc,
        out_shape=jax.ShapeDtypeStruct((BATCH, 2), jnp.float32),
    )(x, sums, tp[0:1], w1t, b1.reshape(1, -1), W2.T, b2.reshape(1, -1))
    return out


# speedup vs baseline: 9.8779x; 1.3145x over previous
import functools

import jax
import jax.numpy as jnp
from jax import lax
from jax.experimental import pallas as pl
from jax.experimental.pallas import tpu as pltpu
from jax.experimental.pallas import tpu_sc as plsc

VOCAB = 100000
EMBED = 100
EPAD = 112
BATCH = 4096
SEQ = 200
NLANES = 16
NVEC = EPAD // NLANES

NC, NS = 2, 16
NW = NC * NS
BPW = BATCH // NW
NCH = 5
CH = SEQ // NCH

VBLK = 4000

_mesh = plsc.VectorSubcoreMesh(core_axis_name="c", subcore_axis_name="s")


@functools.partial(
    pl.kernel,
    out_type=jax.ShapeDtypeStruct((BATCH, EPAD), jnp.float32),
    mesh=_mesh,
    scratch_types=[
        pltpu.VMEM((BPW * SEQ,), jnp.int32),
        pltpu.VMEM((SEQ, EPAD), jnp.float32),
        pltpu.VMEM((SEQ, EPAD), jnp.float32),
        pltpu.VMEM((BPW, EPAD), jnp.float32),
        pltpu.SemaphoreType.DMA,
        pltpu.SemaphoreType.DMA,
        pltpu.SemaphoreType.DMA,
    ],
    compiler_params=pltpu.CompilerParams(use_tc_tiling_on_sc=False),
)
def _pool_sc(x_hbm, tab_hbm, out_hbm, idx_v, rows_a, rows_b, acc_v,
             sem_a, sem_b, sem_i):
    wid = lax.axis_index("s") * NC + lax.axis_index("c")
    base = wid * (BPW * SEQ)
    pltpu.async_copy(x_hbm.at[pl.ds(base, BPW * SEQ)], idx_v, sem_i).wait()

    def fire(r, buf, sem):
        rb = pl.multiple_of(r * SEQ, SEQ)
        for j in range(NCH):
            pltpu.make_async_copy(
                tab_hbm.at[idx_v.at[pl.ds(rb + j * CH, CH)]],
                buf.at[pl.ds(j * CH, CH)],
                sem,
            ).start()

    def drain(r, buf, sem):
        rb = pl.multiple_of(r * SEQ, SEQ)
        for j in range(NCH):
            pltpu.make_async_copy(
                tab_hbm.at[idx_v.at[pl.ds(rb + j * CH, CH)]],
                buf.at[pl.ds(j * CH, CH)],
                sem,
            ).wait()

    def accum(r, buf):
        def body(s, carry):
            return tuple(
                carry[j] + buf[s, pl.ds(j * NLANES, NLANES)]
                for j in range(NVEC)
            )
        acc = lax.fori_loop(
            0, SEQ, body,
            tuple(jnp.zeros((NLANES,), jnp.float32) for _ in range(NVEC)),
        )
        for j in range(NVEC):
            acc_v[r, pl.ds(j * NLANES, NLANES)] = acc[j]

    fire(0, rows_a, sem_a)

    @pl.loop(0, BPW // 2)
    def _(i):
        r0 = i * 2
        r1 = r0 + 1
        fire(r1, rows_b, sem_b)
        drain(r0, rows_a, sem_a)
        accum(r0, rows_a)

        @pl.when(i < BPW // 2 - 1)
        def _():
            fire(r0 + 2, rows_a, sem_a)

        drain(r1, rows_b, sem_b)
        accum(r1, rows_b)

    pltpu.sync_copy(acc_v, out_hbm.at[pl.ds(wid * BPW, BPW)])


def _pad_tc(t_ref, o_ref):
    o_ref[:, :EMBED] = t_ref[...]
    o_ref[:, EMBED:] = jnp.zeros((VBLK, EPAD - EMBED), jnp.float32)


def _mlp_tc(x_ref, sums_ref, t0_ref, w1_ref, b1_ref, w2_ref, b2_ref, out_ref):
    n1 = jnp.sum((x_ref[...] != 0).astype(jnp.float32), axis=1, keepdims=True)
    s = sums_ref[...] - (float(SEQ) - n1) * t0_ref[...]
    h = s / (n1 + 1e-9)
    z = jnp.dot(h, w1_ref[...], preferred_element_type=jnp.float32)
    z = jnp.maximum(z + b1_ref[...], 0.0)
    out_ref[...] = (
        jnp.dot(z, w2_ref[...], preferred_element_type=jnp.float32)
        + b2_ref[...]
    )


@jax.jit
def kernel(x, table, W1, b1, W2, b2):
    tp = pl.pallas_call(
        _pad_tc,
        grid=(VOCAB // VBLK,),
        in_specs=[pl.BlockSpec((VBLK, EMBED), lambda i: (i, 0))],
        out_specs=pl.BlockSpec((VBLK, EPAD), lambda i: (i, 0)),
        out_shape=jax.ShapeDtypeStruct((VOCAB, EPAD), jnp.float32),
    )(table)
    sums = _pool_sc(x.reshape(-1), tp)
    w1t = jnp.pad(W1, ((0, 0), (0, EPAD - EMBED))).T
    out = pl.pallas_call(
        _mlp_tc,
        out_shape=jax.ShapeDtypeStruct((BATCH, 2), jnp.float32),
    )(x, sums, tp[0:1], w1t, b1.reshape(1, -1), W2.T, b2.reshape(1, -1))
    return out


# speedup vs baseline: 10.7584x; 1.0891x over previous
import functools

import jax
import jax.numpy as jnp
from jax import lax
from jax.experimental import pallas as pl
from jax.experimental.pallas import tpu as pltpu
from jax.experimental.pallas import tpu_sc as plsc

VOCAB = 100000
EMBED = 100
EPAD = 128
BATCH = 4096
SEQ = 200
NLANES = 16
NGRP = EPAD // 32

NC, NS = 2, 16
NW = NC * NS
BPW = BATCH // NW
NCH = 5
CH = SEQ // NCH

VBLK = 4000

_mesh = plsc.VectorSubcoreMesh(core_axis_name="c", subcore_axis_name="s")


@functools.partial(
    pl.kernel,
    out_type=jax.ShapeDtypeStruct((BATCH, EPAD), jnp.float32),
    mesh=_mesh,
    scratch_types=[
        pltpu.VMEM((BPW * SEQ,), jnp.int32),
        pltpu.VMEM((SEQ, EPAD), jnp.bfloat16),
        pltpu.VMEM((SEQ, EPAD), jnp.bfloat16),
        pltpu.VMEM((BPW, EPAD), jnp.float32),
        pltpu.SemaphoreType.DMA,
        pltpu.SemaphoreType.DMA,
        pltpu.SemaphoreType.DMA,
    ],
    compiler_params=pltpu.CompilerParams(
        use_tc_tiling_on_sc=False, needs_layout_passes=False
    ),
)
def _pool_sc(x_hbm, tab_hbm, out_hbm, idx_v, rows_a, rows_b, acc_v,
             sem_a, sem_b, sem_i):
    wid = lax.axis_index("s") * NC + lax.axis_index("c")
    base = wid * (BPW * SEQ)
    pltpu.async_copy(x_hbm.at[pl.ds(base, BPW * SEQ)], idx_v, sem_i).wait()

    def fire(r, buf, sem):
        rb = pl.multiple_of(r * SEQ, SEQ)
        for j in range(NCH):
            pltpu.make_async_copy(
                tab_hbm.at[idx_v.at[pl.ds(rb + j * CH, CH)]],
                buf.at[pl.ds(j * CH, CH)],
                sem,
            ).start()

    def drain(r, buf, sem):
        rb = pl.multiple_of(r * SEQ, SEQ)
        for j in range(NCH):
            pltpu.make_async_copy(
                tab_hbm.at[idx_v.at[pl.ds(rb + j * CH, CH)]],
                buf.at[pl.ds(j * CH, CH)],
                sem,
            ).wait()

    def accum(r, buf):
        def body(s, carry):
            new = []
            for g in range(NGRP):
                ev, od = plsc.unpack(
                    buf[s, pl.ds(g * 32, 32)],
                    format=plsc.PackFormat.INTERLEAVED,
                )
                new.append(carry[2 * g] + ev)
                new.append(carry[2 * g + 1] + od)
            return tuple(new)
        acc = lax.fori_loop(
            0, SEQ, body,
            tuple(jnp.zeros((NLANES,), jnp.float32) for _ in range(2 * NGRP)),
        )
        for j in range(2 * NGRP):
            acc_v[r, pl.ds(j * NLANES, NLANES)] = acc[j]

    fire(0, rows_a, sem_a)

    @pl.loop(0, BPW // 2)
    def _(i):
        r0 = i * 2
        r1 = r0 + 1
        fire(r1, rows_b, sem_b)
        drain(r0, rows_a, sem_a)
        accum(r0, rows_a)

        @pl.when(i < BPW // 2 - 1)
        def _():
            fire(r0 + 2, rows_a, sem_a)

        drain(r1, rows_b, sem_b)
        accum(r1, rows_b)

    pltpu.sync_copy(acc_v, out_hbm.at[pl.ds(wid * BPW, BPW)])


def _conv_tc(t_ref, o_ref):
    o_ref[:, :EMBED] = t_ref[...].astype(jnp.bfloat16)
    o_ref[:, EMBED:] = jnp.zeros((VBLK, EPAD - EMBED), jnp.bfloat16)


def _mlp_tc(x_ref, sums_ref, t0_ref, w1_ref, b1_ref, w2_ref, b2_ref, out_ref):
    n1 = jnp.sum((x_ref[...] != 0).astype(jnp.float32), axis=1, keepdims=True)
    s = sums_ref[...] - (float(SEQ) - n1) * t0_ref[...]
    h = s / (n1 + 1e-9)
    z = jnp.dot(h, w1_ref[...], preferred_element_type=jnp.float32)
    z = jnp.maximum(z + b1_ref[...], 0.0)
    out_ref[...] = (
        jnp.dot(z, w2_ref[...], preferred_element_type=jnp.float32)
        + b2_ref[...]
    )


_PERM = tuple(
    32 * (l // 32) + 2 * (l % 16) + (l % 32) // 16 for l in range(EPAD)
)


@jax.jit
def kernel(x, table, W1, b1, W2, b2):
    tb = pl.pallas_call(
        _conv_tc,
        grid=(VOCAB // VBLK,),
        in_specs=[pl.BlockSpec((VBLK, EMBED), lambda i: (i, 0))],
        out_specs=pl.BlockSpec((VBLK, EPAD), lambda i: (i, 0)),
        out_shape=jax.ShapeDtypeStruct((VOCAB, EPAD), jnp.bfloat16),
    )(table)
    sums = _pool_sc(x.reshape(-1), tb)
    perm = jnp.asarray(_PERM, jnp.int32)
    w1p = jnp.pad(W1, ((0, 0), (0, EPAD - EMBED)))
    w1t = w1p.T[perm]
    t0 = jnp.pad(
        table[0].astype(jnp.bfloat16).astype(jnp.float32),
        (0, EPAD - EMBED),
    )[perm].reshape(1, EPAD)
    out = pl.pallas_call(
        _mlp_tc,
        out_shape=jax.ShapeDtypeStruct((BATCH, 2), jnp.float32),
    )(x, sums, t0, w1t, b1.reshape(1, -1), W2.T, b2.reshape(1, -1))
    return out


# speedup vs baseline: 11.8761x; 1.1039x over previous
import functools

import jax
import jax.numpy as jnp
from jax import lax
from jax.experimental import pallas as pl
from jax.experimental.pallas import tpu as pltpu
from jax.experimental.pallas import tpu_sc as plsc

VOCAB = 100000
EMBED = 100
EPAD = 128
BATCH = 4096
SEQ = 200
NLANES = 16
NGRP = EPAD // 32

NC, NS = 2, 16
NW = NC * NS
BPW = BATCH // NW
NCH = 5
CH = SEQ // NCH

VBLK = 2048

_mesh = plsc.VectorSubcoreMesh(core_axis_name="c", subcore_axis_name="s")


@functools.partial(
    pl.kernel,
    out_type=jax.ShapeDtypeStruct((BATCH, EPAD), jnp.float32),
    mesh=_mesh,
    scratch_types=[
        pltpu.VMEM((BPW * SEQ,), jnp.int32),
        pltpu.VMEM((SEQ, EPAD), jnp.bfloat16),
        pltpu.VMEM((SEQ, EPAD), jnp.bfloat16),
        pltpu.VMEM((BPW, EPAD), jnp.float32),
        pltpu.SemaphoreType.DMA,
        pltpu.SemaphoreType.DMA,
        pltpu.SemaphoreType.DMA,
    ],
    compiler_params=pltpu.CompilerParams(
        use_tc_tiling_on_sc=False, needs_layout_passes=False
    ),
)
def _pool_sc(x_hbm, tab_hbm, out_hbm, idx_v, rows_a, rows_b, acc_v,
             sem_a, sem_b, sem_i):
    wid = lax.axis_index("s") * NC + lax.axis_index("c")
    base = wid * (BPW * SEQ)
    pltpu.async_copy(x_hbm.at[pl.ds(base, BPW * SEQ)], idx_v, sem_i).wait()

    def fire(r, buf, sem):
        rb = pl.multiple_of(r * SEQ, SEQ)
        for j in range(NCH):
            pltpu.make_async_copy(
                tab_hbm.at[idx_v.at[pl.ds(rb + j * CH, CH)]],
                buf.at[pl.ds(j * CH, CH)],
                sem,
            ).start()

    def drain(r, buf, sem):
        rb = pl.multiple_of(r * SEQ, SEQ)
        for j in range(NCH):
            pltpu.make_async_copy(
                tab_hbm.at[idx_v.at[pl.ds(rb + j * CH, CH)]],
                buf.at[pl.ds(j * CH, CH)],
                sem,
            ).wait()

    def accum(r, buf):
        def body(s, carry):
            new = []
            for g in range(NGRP):
                ev, od = plsc.unpack(
                    buf[s, pl.ds(g * 32, 32)],
                    format=plsc.PackFormat.INTERLEAVED,
                )
                new.append(carry[2 * g] + ev)
                new.append(carry[2 * g + 1] + od)
            return tuple(new)
        acc = lax.fori_loop(
            0, SEQ, body,
            tuple(jnp.zeros((NLANES,), jnp.float32) for _ in range(2 * NGRP)),
        )
        for j in range(2 * NGRP):
            acc_v[r, pl.ds(j * NLANES, NLANES)] = acc[j]

    fire(0, rows_a, sem_a)

    @pl.loop(0, BPW // 2)
    def _(i):
        r0 = i * 2
        r1 = r0 + 1
        fire(r1, rows_b, sem_b)
        drain(r0, rows_a, sem_a)
        accum(r0, rows_a)

        @pl.when(i < BPW // 2 - 1)
        def _():
            fire(r0 + 2, rows_a, sem_a)

        drain(r1, rows_b, sem_b)
        accum(r1, rows_b)

    pltpu.sync_copy(acc_v, out_hbm.at[pl.ds(wid * BPW, BPW)])


def _conv_tc(t_ref, o_ref):
    o_ref[:, :EMBED] = jnp.transpose(t_ref[...]).astype(jnp.bfloat16)
    o_ref[:, EMBED:] = jnp.zeros((VBLK, EPAD - EMBED), jnp.bfloat16)


def _mlp_tc(x_ref, sums_ref, t0_ref, w1_ref, b1_ref, w2_ref, b2_ref, out_ref):
    n1 = jnp.sum((x_ref[...] != 0).astype(jnp.float32), axis=1, keepdims=True)
    s = sums_ref[...] - (float(SEQ) - n1) * t0_ref[...]
    h = s / (n1 + 1e-9)
    z = jnp.dot(h, w1_ref[...], preferred_element_type=jnp.float32)
    z = jnp.maximum(z + b1_ref[...], 0.0)
    out_ref[...] = (
        jnp.dot(z, w2_ref[...], preferred_element_type=jnp.float32)
        + b2_ref[...]
    )


_PERM = tuple(
    32 * (l // 32) + 2 * (l % 16) + (l % 32) // 16 for l in range(EPAD)
)


@jax.jit
def kernel(x, table, W1, b1, W2, b2):
    tb = pl.pallas_call(
        _conv_tc,
        grid=(pl.cdiv(VOCAB, VBLK),),
        in_specs=[pl.BlockSpec((EMBED, VBLK), lambda i: (0, i))],
        out_specs=pl.BlockSpec((VBLK, EPAD), lambda i: (i, 0)),
        out_shape=jax.ShapeDtypeStruct((VOCAB, EPAD), jnp.bfloat16),
    )(table.T)
    sums = _pool_sc(x.reshape(-1), tb)
    perm = jnp.asarray(_PERM, jnp.int32)
    w1p = jnp.pad(W1, ((0, 0), (0, EPAD - EMBED)))
    w1t = w1p.T[perm]
    t0 = jnp.pad(
        table[0].astype(jnp.bfloat16).astype(jnp.float32),
        (0, EPAD - EMBED),
    )[perm].reshape(1, EPAD)
    out = pl.pallas_call(
        _mlp_tc,
        out_shape=jax.ShapeDtypeStruct((BATCH, 2), jnp.float32),
    )(x, sums, t0, w1t, b1.reshape(1, -1), W2.T, b2.reshape(1, -1))
    return out


# speedup vs baseline: 13.6116x; 1.1461x over previous
import functools

import jax
import jax.numpy as jnp
from jax import lax
from jax.experimental import pallas as pl
from jax.experimental.pallas import tpu as pltpu
from jax.experimental.pallas import tpu_sc as plsc

VOCAB = 100000
EMBED = 100
EPAD = 128
BATCH = 4096
SEQ = 200
NLANES = 16
NGRP = EPAD // 32

NC, NS = 2, 16
NW = NC * NS
BPW = BATCH // NW
NCH = 5
CH = SEQ // NCH

VBLK = 2048

_mesh = plsc.VectorSubcoreMesh(core_axis_name="c", subcore_axis_name="s")


@functools.partial(
    pl.kernel,
    out_type=jax.ShapeDtypeStruct((BATCH, EPAD), jnp.float32),
    mesh=_mesh,
    scratch_types=[
        pltpu.VMEM((BPW * SEQ,), jnp.int32),
        pltpu.VMEM((SEQ, EPAD // 2), jnp.float32),
        pltpu.VMEM((SEQ, EPAD // 2), jnp.float32),
        pltpu.VMEM((BPW, EPAD), jnp.float32),
        pltpu.SemaphoreType.DMA,
        pltpu.SemaphoreType.DMA,
        pltpu.SemaphoreType.DMA,
    ],
    compiler_params=pltpu.CompilerParams(
        use_tc_tiling_on_sc=False, needs_layout_passes=False
    ),
)
def _pool_sc(x_hbm, tab_hbm, out_hbm, idx_v, rows_a, rows_b, acc_v,
             sem_a, sem_b, sem_i):
    wid = lax.axis_index("s") * NC + lax.axis_index("c")
    base = wid * (BPW * SEQ)
    pltpu.async_copy(x_hbm.at[pl.ds(base, BPW * SEQ)], idx_v, sem_i).wait()

    def fire(r, buf, sem):
        rb = pl.multiple_of(r * SEQ, SEQ)
        for j in range(NCH):
            pltpu.make_async_copy(
                tab_hbm.at[idx_v.at[pl.ds(rb + j * CH, CH)]],
                buf.at[pl.ds(j * CH, CH)],
                sem,
            ).start()

    def drain(r, buf, sem):
        rb = pl.multiple_of(r * SEQ, SEQ)
        for j in range(NCH):
            pltpu.make_async_copy(
                tab_hbm.at[idx_v.at[pl.ds(rb + j * CH, CH)]],
                buf.at[pl.ds(j * CH, CH)],
                sem,
            ).wait()

    def accum(r, buf):
        def body(s, carry):
            new = []
            for g in range(NGRP):
                pair = plsc.bitcast(
                    buf[s, pl.ds(g * NLANES, NLANES)], jnp.bfloat16
                )
                ev, od = plsc.unpack(
                    pair, format=plsc.PackFormat.INTERLEAVED
                )
                new.append(carry[2 * g] + ev)
                new.append(carry[2 * g + 1] + od)
            return tuple(new)
        acc = lax.fori_loop(
            0, SEQ, body,
            tuple(jnp.zeros((NLANES,), jnp.float32) for _ in range(2 * NGRP)),
        )
        for j in range(2 * NGRP):
            acc_v[r, pl.ds(j * NLANES, NLANES)] = acc[j]

    fire(0, rows_a, sem_a)

    @pl.loop(0, BPW // 2)
    def _(i):
        r0 = i * 2
        r1 = r0 + 1
        fire(r1, rows_b, sem_b)
        drain(r0, rows_a, sem_a)
        accum(r0, rows_a)

        @pl.when(i < BPW // 2 - 1)
        def _():
            fire(r0 + 2, rows_a, sem_a)

        drain(r1, rows_b, sem_b)
        accum(r1, rows_b)

    pltpu.sync_copy(acc_v, out_hbm.at[pl.ds(wid * BPW, BPW)])


def _conv_tc(t_ref, o_ref):
    b = t_ref[...].astype(jnp.bfloat16)
    bp = jnp.concatenate(
        [b, jnp.zeros((EPAD - EMBED, VBLK), jnp.bfloat16)], axis=0)
    w = pltpu.bitcast(bp, jnp.float32)
    o_ref[...] = jnp.transpose(w)


def _mlp_tc(x_ref, sums_ref, t0_ref, w1_ref, b1_ref, w2_ref, b2_ref, out_ref):
    n1 = jnp.sum((x_ref[...] != 0).astype(jnp.float32), axis=1, keepdims=True)
    s = sums_ref[...] - (float(SEQ) - n1) * t0_ref[...]
    h = s / (n1 + 1e-9)
    z = jnp.dot(h, w1_ref[...], preferred_element_type=jnp.float32)
    z = jnp.maximum(z + b1_ref[...], 0.0)
    out_ref[...] = (
        jnp.dot(z, w2_ref[...], preferred_element_type=jnp.float32)
        + b2_ref[...]
    )


_PERM = tuple(
    32 * (l // 32) + 2 * (l % 16) + (l % 32) // 16 for l in range(EPAD)
)


@jax.jit
def kernel(x, table, W1, b1, W2, b2):
    tb = pl.pallas_call(
        _conv_tc,
        grid=(pl.cdiv(VOCAB, VBLK),),
        in_specs=[pl.BlockSpec((EMBED, VBLK), lambda i: (0, i))],
        out_specs=pl.BlockSpec((VBLK, EPAD // 2), lambda i: (i, 0)),
        out_shape=jax.ShapeDtypeStruct((VOCAB, EPAD // 2), jnp.float32),
    )(table.T)
    sums = _pool_sc(x.reshape(-1), tb)
    perm = jnp.asarray(_PERM, jnp.int32)
    w1p = jnp.pad(W1, ((0, 0), (0, EPAD - EMBED)))
    w1t = w1p.T[perm]
    t0 = jnp.pad(
        table[0].astype(jnp.bfloat16).astype(jnp.float32),
        (0, EPAD - EMBED),
    )[perm].reshape(1, EPAD)
    out = pl.pallas_call(
        _mlp_tc,
        out_shape=jax.ShapeDtypeStruct((BATCH, 2), jnp.float32),
    )(x, sums, t0, w1t, b1.reshape(1, -1), W2.T, b2.reshape(1, -1))
    return out


# speedup vs baseline: 13.8105x; 1.0146x over previous
import functools

import jax
import jax.numpy as jnp
from jax import lax
from jax.experimental import pallas as pl
from jax.experimental.pallas import tpu as pltpu
from jax.experimental.pallas import tpu_sc as plsc

VOCAB = 100000
EMBED = 100
EPAD = 128
BATCH = 4096
SEQ = 200
NLANES = 16
NGRP = EPAD // 32

NC, NS = 2, 16
NW = NC * NS
BPW = BATCH // NW
NCH = 5
CH = SEQ // NCH

VBLK = 2048

_mesh = plsc.VectorSubcoreMesh(core_axis_name="c", subcore_axis_name="s")


@functools.partial(
    pl.kernel,
    out_type=jax.ShapeDtypeStruct((BATCH, EPAD), jnp.float32),
    mesh=_mesh,
    scratch_types=[
        pltpu.VMEM((BPW * SEQ,), jnp.int32),
        pltpu.VMEM((SEQ, EPAD // 2), jnp.float32),
        pltpu.VMEM((SEQ, EPAD // 2), jnp.float32),
        pltpu.VMEM((BPW, EPAD), jnp.float32),
        pltpu.SemaphoreType.DMA,
        pltpu.SemaphoreType.DMA,
        pltpu.SemaphoreType.DMA,
    ],
    compiler_params=pltpu.CompilerParams(
        use_tc_tiling_on_sc=False, needs_layout_passes=False
    ),
)
def _pool_sc(x_hbm, tab_hbm, out_hbm, idx_v, rows_a, rows_b, acc_v,
             sem_a, sem_b, sem_i):
    wid = lax.axis_index("s") * NC + lax.axis_index("c")
    base = wid * (BPW * SEQ)
    pltpu.async_copy(x_hbm.at[pl.ds(base, BPW * SEQ)], idx_v, sem_i).wait()

    def fire(r, buf, sem):
        rb = pl.multiple_of(r * SEQ, SEQ)
        for j in range(NCH):
            pltpu.make_async_copy(
                tab_hbm.at[idx_v.at[pl.ds(rb + j * CH, CH)]],
                buf.at[pl.ds(j * CH, CH)],
                sem,
            ).start()

    def drain(r, buf, sem):
        rb = pl.multiple_of(r * SEQ, SEQ)
        for j in range(NCH):
            pltpu.make_async_copy(
                tab_hbm.at[idx_v.at[pl.ds(rb + j * CH, CH)]],
                buf.at[pl.ds(j * CH, CH)],
                sem,
            ).wait()

    def accum(r, buf):
        def body(s, carry):
            new = []
            for g in range(NGRP):
                pair = plsc.bitcast(
                    buf[s, pl.ds(g * NLANES, NLANES)], jnp.bfloat16
                )
                ev, od = plsc.unpack(
                    pair, format=plsc.PackFormat.INTERLEAVED
                )
                new.append(carry[2 * g] + ev)
                new.append(carry[2 * g + 1] + od)
            return tuple(new)
        acc = lax.fori_loop(
            0, SEQ, body,
            tuple(jnp.zeros((NLANES,), jnp.float32) for _ in range(2 * NGRP)),
            unroll=4,
        )
        for j in range(2 * NGRP):
            acc_v[r, pl.ds(j * NLANES, NLANES)] = acc[j]

    fire(0, rows_a, sem_a)

    @pl.loop(0, BPW // 2)
    def _(i):
        r0 = i * 2
        r1 = r0 + 1
        fire(r1, rows_b, sem_b)
        drain(r0, rows_a, sem_a)
        accum(r0, rows_a)

        @pl.when(i < BPW // 2 - 1)
        def _():
            fire(r0 + 2, rows_a, sem_a)

        drain(r1, rows_b, sem_b)
        accum(r1, rows_b)

    pltpu.sync_copy(acc_v, out_hbm.at[pl.ds(wid * BPW, BPW)])


def _conv_tc(t_ref, o_ref):
    b = t_ref[...].astype(jnp.bfloat16)
    bp = jnp.concatenate(
        [b, jnp.zeros((EPAD - EMBED, VBLK), jnp.bfloat16)], axis=0)
    w = pltpu.bitcast(bp, jnp.float32)
    o_ref[...] = jnp.transpose(w)


def _mlp_tc(x_ref, sums_ref, t0_ref, w1_ref, b1_ref, w2_ref, b2_ref, out_ref):
    n1 = jnp.sum((x_ref[...] != 0).astype(jnp.float32), axis=1, keepdims=True)
    s = sums_ref[...] - (float(SEQ) - n1) * t0_ref[...]
    h = s / (n1 + 1e-9)
    z = jnp.dot(h, w1_ref[...], preferred_element_type=jnp.float32)
    z = jnp.maximum(z + b1_ref[...], 0.0)
    out_ref[...] = (
        jnp.dot(z, w2_ref[...], preferred_element_type=jnp.float32)
        + b2_ref[...]
    )


_PERM = tuple(
    32 * (l // 32) + 2 * (l % 16) + (l % 32) // 16 for l in range(EPAD)
)


@jax.jit
def kernel(x, table, W1, b1, W2, b2):
    tb = pl.pallas_call(
        _conv_tc,
        grid=(pl.cdiv(VOCAB, VBLK),),
        in_specs=[pl.BlockSpec((EMBED, VBLK), lambda i: (0, i))],
        out_specs=pl.BlockSpec((VBLK, EPAD // 2), lambda i: (i, 0)),
        out_shape=jax.ShapeDtypeStruct((VOCAB, EPAD // 2), jnp.float32),
    )(table.T)
    sums = _pool_sc(x.reshape(-1), tb)
    perm = jnp.asarray(_PERM, jnp.int32)
    w1p = jnp.pad(W1, ((0, 0), (0, EPAD - EMBED)))
    w1t = w1p.T[perm]
    t0 = jnp.pad(
        table[0].astype(jnp.bfloat16).astype(jnp.float32),
        (0, EPAD - EMBED),
    )[perm].reshape(1, EPAD)
    out = pl.pallas_call(
        _mlp_tc,
        out_shape=jax.ShapeDtypeStruct((BATCH, 2), jnp.float32),
    )(x, sums, t0, w1t, b1.reshape(1, -1), W2.T, b2.reshape(1, -1))
    return out


# speedup vs baseline: 14.6925x; 1.0639x over previous
import functools

import jax
import jax.numpy as jnp
from jax import lax
from jax.experimental import pallas as pl
from jax.experimental.pallas import tpu as pltpu
from jax.experimental.pallas import tpu_sc as plsc

VOCAB = 100000
EMBED = 100
EPAD = 128
BATCH = 4096
SEQ = 200
NLANES = 16
NGRP = EPAD // 32

NC, NS = 2, 16
NW = NC * NS
BPW = BATCH // NW
NCH = 5
CH = SEQ // NCH

VBLK = 4096

_mesh = plsc.VectorSubcoreMesh(core_axis_name="c", subcore_axis_name="s")


@functools.partial(
    pl.kernel,
    out_type=jax.ShapeDtypeStruct((BATCH, EPAD), jnp.float32),
    mesh=_mesh,
    scratch_types=[
        pltpu.VMEM((BPW * SEQ,), jnp.int32),
        pltpu.VMEM((SEQ, EPAD // 2), jnp.float32),
        pltpu.VMEM((SEQ, EPAD // 2), jnp.float32),
        pltpu.VMEM((BPW, EPAD), jnp.float32),
        pltpu.SemaphoreType.DMA,
        pltpu.SemaphoreType.DMA,
        pltpu.SemaphoreType.DMA,
    ],
    compiler_params=pltpu.CompilerParams(
        use_tc_tiling_on_sc=False, needs_layout_passes=False
    ),
)
def _pool_sc(x_hbm, tab_hbm, out_hbm, idx_v, rows_a, rows_b, acc_v,
             sem_a, sem_b, sem_i):
    wid = lax.axis_index("s") * NC + lax.axis_index("c")
    base = wid * (BPW * SEQ)
    pltpu.async_copy(x_hbm.at[pl.ds(base, BPW * SEQ)], idx_v, sem_i).wait()

    def fire(r, buf, sem):
        rb = pl.multiple_of(r * SEQ, SEQ)
        for j in range(NCH):
            pltpu.make_async_copy(
                tab_hbm.at[idx_v.at[pl.ds(rb + j * CH, CH)]],
                buf.at[pl.ds(j * CH, CH)],
                sem,
            ).start()

    def drain(r, buf, sem):
        rb = pl.multiple_of(r * SEQ, SEQ)
        for j in range(NCH):
            pltpu.make_async_copy(
                tab_hbm.at[idx_v.at[pl.ds(rb + j * CH, CH)]],
                buf.at[pl.ds(j * CH, CH)],
                sem,
            ).wait()

    def accum(r, buf):
        def body(s, carry):
            new = []
            for g in range(NGRP):
                pair = plsc.bitcast(
                    buf[s, pl.ds(g * NLANES, NLANES)], jnp.bfloat16
                )
                ev, od = plsc.unpack(
                    pair, format=plsc.PackFormat.INTERLEAVED
                )
                new.append(carry[2 * g] + ev)
                new.append(carry[2 * g + 1] + od)
            return tuple(new)
        acc = lax.fori_loop(
            0, SEQ, body,
            tuple(jnp.zeros((NLANES,), jnp.float32) for _ in range(2 * NGRP)),
            unroll=4,
        )
        for j in range(2 * NGRP):
            acc_v[r, pl.ds(j * NLANES, NLANES)] = acc[j]

    fire(0, rows_a, sem_a)

    @pl.loop(0, BPW // 2)
    def _(i):
        r0 = i * 2
        r1 = r0 + 1
        fire(r1, rows_b, sem_b)
        drain(r0, rows_a, sem_a)
        accum(r0, rows_a)

        @pl.when(i < BPW // 2 - 1)
        def _():
            fire(r0 + 2, rows_a, sem_a)

        drain(r1, rows_b, sem_b)
        accum(r1, rows_b)

    pltpu.sync_copy(acc_v, out_hbm.at[pl.ds(wid * BPW, BPW)])


def _conv_tc(t_ref, o_ref):
    b = t_ref[...].astype(jnp.bfloat16)
    bp = jnp.concatenate(
        [b, jnp.zeros((EPAD - EMBED, VBLK), jnp.bfloat16)], axis=0)
    w = pltpu.bitcast(bp, jnp.float32)
    o_ref[...] = jnp.transpose(w)


def _mlp_tc(x_ref, sums_ref, t0_ref, w1_ref, b1_ref, w2_ref, b2_ref, out_ref):
    n1 = jnp.sum((x_ref[...] != 0).astype(jnp.float32), axis=1, keepdims=True)
    s = sums_ref[...] - (float(SEQ) - n1) * t0_ref[...]
    h = s / (n1 + 1e-9)
    z = jnp.dot(h, w1_ref[...], preferred_element_type=jnp.float32)
    z = jnp.maximum(z + b1_ref[...], 0.0)
    out_ref[...] = (
        jnp.dot(z, w2_ref[...], preferred_element_type=jnp.float32)
        + b2_ref[...]
    )


_PERM = tuple(
    32 * (l // 32) + 2 * (l % 16) + (l % 32) // 16 for l in range(EPAD)
)


@jax.jit
def kernel(x, table, W1, b1, W2, b2):
    tb = pl.pallas_call(
        _conv_tc,
        grid=(pl.cdiv(VOCAB, VBLK),),
        in_specs=[pl.BlockSpec((EMBED, VBLK), lambda i: (0, i))],
        out_specs=pl.BlockSpec((VBLK, EPAD // 2), lambda i: (i, 0)),
        out_shape=jax.ShapeDtypeStruct((VOCAB, EPAD // 2), jnp.float32),
    )(table.T)
    sums = _pool_sc(x.reshape(-1), tb)
    perm = jnp.asarray(_PERM, jnp.int32)
    w1p = jnp.pad(W1, ((0, 0), (0, EPAD - EMBED)))
    w1t = w1p.T[perm]
    t0 = jnp.pad(
        table[0].astype(jnp.bfloat16).astype(jnp.float32),
        (0, EPAD - EMBED),
    )[perm].reshape(1, EPAD)
    out = pl.pallas_call(
        _mlp_tc,
        out_shape=jax.ShapeDtypeStruct((BATCH, 2), jnp.float32),
    )(x, sums, t0, w1t, b1.reshape(1, -1), W2.T, b2.reshape(1, -1))
    return out


# speedup vs baseline: 15.1328x; 1.0300x over previous
import functools

import jax
import jax.numpy as jnp
from jax import lax
from jax.experimental import pallas as pl
from jax.experimental.pallas import tpu as pltpu
from jax.experimental.pallas import tpu_sc as plsc

VOCAB = 100000
EMBED = 100
EPAD = 128
BATCH = 4096
SEQ = 200
NLANES = 16
NGRP = EPAD // 32

NC, NS = 2, 16
NW = NC * NS
BPW = BATCH // NW
CHUNKS = ((0, 104), (104, 96))

VBLK = 8192

_mesh = plsc.VectorSubcoreMesh(core_axis_name="c", subcore_axis_name="s")


@functools.partial(
    pl.kernel,
    out_type=jax.ShapeDtypeStruct((BATCH, EPAD), jnp.float32),
    mesh=_mesh,
    scratch_types=[
        pltpu.VMEM((BPW * SEQ,), jnp.int32),
        pltpu.VMEM((SEQ, EPAD // 2), jnp.float32),
        pltpu.VMEM((SEQ, EPAD // 2), jnp.float32),
        pltpu.VMEM((BPW, EPAD), jnp.float32),
        pltpu.SemaphoreType.DMA,
        pltpu.SemaphoreType.DMA,
        pltpu.SemaphoreType.DMA,
    ],
    compiler_params=pltpu.CompilerParams(
        use_tc_tiling_on_sc=False, needs_layout_passes=False
    ),
)
def _pool_sc(x_hbm, tab_hbm, out_hbm, idx_v, rows_a, rows_b, acc_v,
             sem_a, sem_b, sem_i):
    wid = lax.axis_index("s") * NC + lax.axis_index("c")
    base = wid * (BPW * SEQ)
    pltpu.async_copy(x_hbm.at[pl.ds(base, BPW * SEQ)], idx_v, sem_i).wait()

    def fire(r, buf, sem):
        rb = pl.multiple_of(r * SEQ, SEQ)
        for off, n in CHUNKS:
            pltpu.make_async_copy(
                tab_hbm.at[idx_v.at[pl.ds(rb + off, n)]],
                buf.at[pl.ds(off, n)],
                sem,
            ).start()

    def drain(r, buf, sem):
        rb = pl.multiple_of(r * SEQ, SEQ)
        for off, n in CHUNKS:
            pltpu.make_async_copy(
                tab_hbm.at[idx_v.at[pl.ds(rb + off, n)]],
                buf.at[pl.ds(off, n)],
                sem,
            ).wait()

    def accum(r, buf):
        def body(s, carry):
            new = []
            for g in range(NGRP):
                pair = plsc.bitcast(
                    buf[s, pl.ds(g * NLANES, NLANES)], jnp.bfloat16
                )
                ev, od = plsc.unpack(
                    pair, format=plsc.PackFormat.INTERLEAVED
                )
                new.append(carry[2 * g] + ev)
                new.append(carry[2 * g + 1] + od)
            return tuple(new)
        acc = lax.fori_loop(
            0, SEQ, body,
            tuple(jnp.zeros((NLANES,), jnp.float32) for _ in range(2 * NGRP)),
            unroll=4,
        )
        for j in range(2 * NGRP):
            acc_v[r, pl.ds(j * NLANES, NLANES)] = acc[j]

    fire(0, rows_a, sem_a)

    @pl.loop(0, BPW // 2)
    def _(i):
        r0 = i * 2
        r1 = r0 + 1
        fire(r1, rows_b, sem_b)
        drain(r0, rows_a, sem_a)
        accum(r0, rows_a)

        @pl.when(i < BPW // 2 - 1)
        def _():
            fire(r0 + 2, rows_a, sem_a)

        drain(r1, rows_b, sem_b)
        accum(r1, rows_b)

    pltpu.sync_copy(acc_v, out_hbm.at[pl.ds(wid * BPW, BPW)])


def _conv_tc(t_ref, o_ref):
    b = t_ref[...].astype(jnp.bfloat16)
    bp = jnp.concatenate(
        [b, jnp.zeros((EPAD - EMBED, VBLK), jnp.bfloat16)], axis=0)
    w = pltpu.bitcast(bp, jnp.float32)
    o_ref[...] = jnp.transpose(w)


def _mlp_tc(x_ref, sums_ref, t0_ref, w1_ref, b1_ref, w2_ref, b2_ref, out_ref):
    n1 = jnp.sum((x_ref[...] != 0).astype(jnp.float32), axis=1, keepdims=True)
    s = sums_ref[...] - (float(SEQ) - n1) * t0_ref[...]
    h = s / (n1 + 1e-9)
    z = jnp.dot(h, w1_ref[...], preferred_element_type=jnp.float32)
    z = jnp.maximum(z + b1_ref[...], 0.0)
    out_ref[...] = (
        jnp.dot(z, w2_ref[...], preferred_element_type=jnp.float32)
        + b2_ref[...]
    )


_PERM = tuple(
    32 * (l // 32) + 2 * (l % 16) + (l % 32) // 16 for l in range(EPAD)
)


@jax.jit
def kernel(x, table, W1, b1, W2, b2):
    tb = pl.pallas_call(
        _conv_tc,
        grid=(pl.cdiv(VOCAB, VBLK),),
        in_specs=[pl.BlockSpec((EMBED, VBLK), lambda i: (0, i))],
        out_specs=pl.BlockSpec((VBLK, EPAD // 2), lambda i: (i, 0)),
        out_shape=jax.ShapeDtypeStruct((VOCAB, EPAD // 2), jnp.float32),
    )(table.T)
    sums = _pool_sc(x.reshape(-1), tb)
    perm = jnp.asarray(_PERM, jnp.int32)
    w1p = jnp.pad(W1, ((0, 0), (0, EPAD - EMBED)))
    w1t = w1p.T[perm]
    t0 = jnp.pad(
        table[0].astype(jnp.bfloat16).astype(jnp.float32),
        (0, EPAD - EMBED),
    )[perm].reshape(1, EPAD)
    out = pl.pallas_call(
        _mlp_tc,
        out_shape=jax.ShapeDtypeStruct((BATCH, 2), jnp.float32),
    )(x, sums, t0, w1t, b1.reshape(1, -1), W2.T, b2.reshape(1, -1))
    return out
